# baseline (speedup 1.0000x reference)
import functools

import jax
import jax.numpy as jnp
from jax import lax
from jax.experimental import pallas as pl
from jax.experimental.pallas import tpu as pltpu
from jax.experimental.pallas import tpu_sc as plsc

NC = 2
NS = 16
LANE = 128


def _edge_geometry(E):
    rt = -(-E // (LANE * NC * NS))
    rt = -(-rt // 16) * 16
    return rt, 16


def _sc_mesh():
    return plsc.VectorSubcoreMesh(
        core_axis_name="c", subcore_axis_name="s", num_cores=NC, num_subcores=NS
    )


def _make_deg_kernel(n_pad, r_t, g_chunk):
    nslice = n_pad // NS

    @functools.partial(
        pl.kernel,
        out_type=jax.ShapeDtypeStruct((NC, n_pad), jnp.float32),
        mesh=_sc_mesh(),
        compiler_params=pltpu.CompilerParams(use_tc_tiling_on_sc=False),
        scratch_types=[
            pltpu.VMEM((g_chunk, LANE), jnp.int32),
            pltpu.VMEM((LANE,), jnp.float32),
            pltpu.VMEM((nslice,), jnp.float32),
            pltpu.SemaphoreType.DMA,
            pltpu.VMEM_SHARED((n_pad,), jnp.float32),
        ],
    )
    def deg_kernel(dstm, zeros1, degp, dstb, onesb, bounce, ssem, shacc):
        c = lax.axis_index("c")
        s = lax.axis_index("s")
        t = c * NS + s
        sl = pl.ds(s * nslice, nslice)
        pltpu.sync_copy(zeros1.at[sl], bounce)
        pltpu.sync_copy(bounce, shacc.at[sl])
        for i in range(LANE // 16):
            onesb[pl.ds(i * 16, 16)] = jnp.ones((16,), jnp.float32)
        plsc.subcore_barrier()
        row0 = t * r_t

        def chunk(i, carry):
            base = row0 + i * g_chunk
            pltpu.sync_copy(dstm.at[pl.ds(base, g_chunk)], dstb)
            sd = [
                pltpu.async_copy(onesb, shacc.at[dstb.at[g]], ssem, add=True)
                for g in range(g_chunk)
            ]
            for d in sd:
                d.wait()
            return carry

        lax.fori_loop(0, r_t // g_chunk, chunk, 0)
        plsc.subcore_barrier()
        pltpu.sync_copy(shacc.at[sl], bounce)
        pltpu.sync_copy(bounce, degp.at[c].at[sl])

    return deg_kernel


def _make_edge_kernel(n_pad, F, r_t, g_chunk):
    nslice = n_pad // NS

    @functools.partial(
        pl.kernel,
        out_type=jax.ShapeDtypeStruct((NC, n_pad, F), jnp.float32),
        mesh=_sc_mesh(),
        compiler_params=pltpu.CompilerParams(use_tc_tiling_on_sc=False),
        scratch_types=[
            pltpu.VMEM((g_chunk, LANE), jnp.int32),
            pltpu.VMEM((g_chunk, LANE), jnp.int32),
            pltpu.VMEM((g_chunk, LANE, F), jnp.float32),
            pltpu.VMEM((nslice // 4, F), jnp.float32),
            pltpu.SemaphoreType.DMA,
            pltpu.SemaphoreType.DMA,
            pltpu.VMEM_SHARED((n_pad, F), jnp.float32),
        ],
    )
    def edge_kernel(gtab, srcm, dstm, zeros, outp, srcb, dstb, rowb, bounce,
                    gsem, ssem, shacc):
        c = lax.axis_index("c")
        s = lax.axis_index("s")
        t = c * NS + s
        q = nslice // 4
        for j in range(4):
            qs = pl.ds(s * nslice + j * q, q)
            pltpu.sync_copy(zeros.at[qs], bounce)
            pltpu.sync_copy(bounce, shacc.at[qs])
        plsc.subcore_barrier()
        row0 = t * r_t

        def chunk(i, carry):
            base = row0 + i * g_chunk
            pltpu.sync_copy(srcm.at[pl.ds(base, g_chunk)], srcb)
            pltpu.sync_copy(dstm.at[pl.ds(base, g_chunk)], dstb)
            gd = [
                pltpu.async_copy(gtab.at[srcb.at[g]], rowb.at[g], gsem)
                for g in range(g_chunk)
            ]
            for d in gd:
                d.wait()
            sd = [
                pltpu.async_copy(rowb.at[g], shacc.at[dstb.at[g]], ssem, add=True)
                for g in range(g_chunk)
            ]
            for d in sd:
                d.wait()
            return carry

        lax.fori_loop(0, r_t // g_chunk, chunk, 0)
        plsc.subcore_barrier()
        for j in range(4):
            qs = pl.ds(s * nslice + j * q, q)
            pltpu.sync_copy(shacc.at[qs], bounce)
            pltpu.sync_copy(bounce, outp.at[c].at[qs])

    return edge_kernel


def _make_edge2_kernel(n_pad, F, r_t, g_chunk):
    nslice = n_pad // NS

    @functools.partial(
        pl.kernel,
        out_type=jax.ShapeDtypeStruct((2, NC, n_pad, F), jnp.float32),
        mesh=_sc_mesh(),
        compiler_params=pltpu.CompilerParams(use_tc_tiling_on_sc=False),
        scratch_types=[
            pltpu.VMEM((g_chunk, LANE), jnp.int32),
            pltpu.VMEM((g_chunk, LANE), jnp.int32),
            pltpu.VMEM((g_chunk, LANE, F), jnp.float32),
            pltpu.VMEM((nslice // 4, F), jnp.float32),
            pltpu.SemaphoreType.DMA,
            pltpu.SemaphoreType.DMA,
            pltpu.VMEM_SHARED((n_pad, F), jnp.float32),
        ],
    )
    def edge2_kernel(tabA, tabB, srcm, dstm, zeros, outp, srcb, dstb, rowb,
                     bounce, gsem, ssem, shacc):
        c = lax.axis_index("c")
        s = lax.axis_index("s")
        t = c * NS + s
        q = nslice // 4
        row0 = t * r_t

        for phase, tab in ((0, tabA), (1, tabB)):
            for j in range(4):
                qs = pl.ds(s * nslice + j * q, q)
                pltpu.sync_copy(zeros.at[qs], bounce)
                pltpu.sync_copy(bounce, shacc.at[qs])
            plsc.subcore_barrier()

            def chunk(i, carry):
                base = row0 + i * g_chunk
                pltpu.sync_copy(srcm.at[pl.ds(base, g_chunk)], srcb)
                pltpu.sync_copy(dstm.at[pl.ds(base, g_chunk)], dstb)
                gd = [
                    pltpu.async_copy(tab.at[srcb.at[g]], rowb.at[g], gsem)
                    for g in range(g_chunk)
                ]
                for d in gd:
                    d.wait()
                sd = [
                    pltpu.async_copy(rowb.at[g], shacc.at[dstb.at[g]], ssem,
                                     add=True)
                    for g in range(g_chunk)
                ]
                for d in sd:
                    d.wait()
                return carry

            lax.fori_loop(0, r_t // g_chunk, chunk, 0)
            plsc.subcore_barrier()
            for j in range(4):
                qs = pl.ds(s * nslice + j * q, q)
                pltpu.sync_copy(shacc.at[qs], bounce)
                pltpu.sync_copy(bounce, outp.at[phase].at[c].at[qs])
            plsc.subcore_barrier()

    return edge2_kernel


def _row_spec(R, F):
    return pl.BlockSpec((R, F), lambda i: (i, 0))


def _full_spec(shape):
    return pl.BlockSpec(shape, lambda i: tuple(0 for _ in shape))


def _tc1(degp0, degp1, x_pad, W1, n_pad, R):
    def body(p0, p1, xr, w1, dinv_o, y1_o):
        deg = p0[...] + p1[...] + 1.0
        dinv = lax.rsqrt(deg)
        dinv_o[...] = dinv
        y1_o[...] = dinv * jnp.dot(xr[...], w1[...])

    return pl.pallas_call(
        body,
        grid=(n_pad // R,),
        in_specs=[_row_spec(R, 1), _row_spec(R, 1), _row_spec(R, 2),
                  _full_spec((2, 16))],
        out_specs=[_row_spec(R, 1), _row_spec(R, 16)],
        out_shape=[
            jax.ShapeDtypeStruct((n_pad, 1), jnp.float32),
            jax.ShapeDtypeStruct((n_pad, 16), jnp.float32),
        ],
    )(degp0, degp1, x_pad, W1)


def _tc2(dinv, y1, a0, a1, W2, b1, n_pad, R):
    def body(dv, y1r, a0r, a1r, w2a, w2b, b1r, y2a_o, y2b_o):
        d = dv[...]
        h1 = jnp.maximum(d * (a0r[...] + a1r[...] + y1r[...]) + b1r[...], 0.0)
        y2a_o[...] = d * jnp.dot(h1, w2a[...])
        y2b_o[...] = d * jnp.dot(h1, w2b[...])

    return pl.pallas_call(
        body,
        grid=(n_pad // R,),
        in_specs=[
            _row_spec(R, 1),
            _row_spec(R, 16),
            _row_spec(R, 16),
            _row_spec(R, 16),
            _full_spec((16, 16)),
            _full_spec((16, 16)),
            _full_spec((1, 16)),
        ],
        out_specs=[_row_spec(R, 16), _row_spec(R, 16)],
        out_shape=[
            jax.ShapeDtypeStruct((n_pad, 16), jnp.float32),
            jax.ShapeDtypeStruct((n_pad, 16), jnp.float32),
        ],
    )(dinv, y1, a0, a1, W2[:, :16], W2[:, 16:], b1.reshape(1, 16))


def _tc3(dinv, y2a, y2b, aA0, aA1, aB0, aB1, b2, W3, b3, W4, b4, n_pad, R):
    def body(dv, y2ar, y2br, a0r, a1r, b0r, b1r_, b2r, w3, b3r, w4, b4r, out_o):
        d = dv[...]
        za = d * (a0r[...] + a1r[...] + y2ar[...])
        zb = d * (b0r[...] + b1r_[...] + y2br[...])
        h2 = jnp.maximum(jnp.concatenate([za, zb], axis=1) + b2r[...], 0.0)
        h3 = jnp.dot(h2, w3[...]) + b3r[...]
        out_o[...] = jnp.dot(h3, w4[...]) + b4r[...]

    return pl.pallas_call(
        body,
        grid=(n_pad // R,),
        in_specs=[
            _row_spec(R, 1),
            _row_spec(R, 16),
            _row_spec(R, 16),
            _row_spec(R, 16),
            _row_spec(R, 16),
            _row_spec(R, 16),
            _row_spec(R, 16),
            _full_spec((1, 32)),
            _full_spec((32, 16)),
            _full_spec((1, 16)),
            _full_spec((16, 1)),
            _full_spec((1, 1)),
        ],
        out_specs=_row_spec(R, 1),
        out_shape=jax.ShapeDtypeStruct((n_pad, 1), jnp.float32),
    )(dinv, y2a, y2b, aA0, aA1, aB0, aB1, b2.reshape(1, 32), W3,
      b3.reshape(1, 16), W4, b4.reshape(1, 1))


def kernel(x, edge_index, W1, b1, W2, b2, W3, b3, W4, b4):
    N = x.shape[0]
    E = edge_index.shape[1]
    n_pad = -(-N // (NS * LANE)) * (NS * LANE)
    if n_pad == N:
        n_pad += NS * LANE
    r_t, g_chunk = _edge_geometry(E)
    e_pad = r_t * LANE * NC * NS

    src = edge_index[0]
    dst = edge_index[1]
    pad = e_pad - E
    if pad:
        ar = jnp.arange(pad, dtype=edge_index.dtype)
        src = jnp.concatenate([src, ar % N])
        dst = jnp.concatenate([dst, N + ar % (n_pad - N)])
    srcm = src.reshape(-1, LANE)
    dstm = dst.reshape(-1, LANE)

    x_pad = jnp.zeros((n_pad, 2), jnp.float32).at[:N].set(x)
    zeros1 = jnp.zeros((n_pad,), jnp.float32)
    zeros16 = jnp.zeros((n_pad, 16), jnp.float32)

    degp = _make_deg_kernel(n_pad, r_t, g_chunk)(dstm, zeros1)
    R = n_pad // 16
    dinv, y1 = _tc1(
        degp[0].reshape(n_pad, 1), degp[1].reshape(n_pad, 1), x_pad, W1, n_pad, R
    )

    edge16 = _make_edge_kernel(n_pad, 16, r_t, g_chunk)
    a1p = edge16(y1, srcm, dstm, zeros16)
    y2a, y2b = _tc2(dinv, y1, a1p[0], a1p[1], W2, b1, n_pad, R)

    a2 = _make_edge2_kernel(n_pad, 16, r_t, g_chunk)(y2a, y2b, srcm, dstm,
                                                     zeros16)
    out = _tc3(dinv, y2a, y2b, a2[0, 0], a2[0, 1], a2[1, 0], a2[1, 1], b2,
               W3, b3, W4, b4, n_pad, R)

    return out[:N]


# speedup vs baseline: 1.0447x; 1.0447x over previous
import functools

import jax
import jax.numpy as jnp
from jax import lax
from jax.experimental import pallas as pl
from jax.experimental.pallas import tpu as pltpu
from jax.experimental.pallas import tpu_sc as plsc

NC = 2
NS = 16
LANE = 128


def _edge_geometry(E):
    rt = -(-E // (LANE * NC * NS))
    rt = -(-rt // 16) * 16
    return rt, 16


def _sc_mesh():
    return plsc.VectorSubcoreMesh(
        core_axis_name="c", subcore_axis_name="s", num_cores=NC, num_subcores=NS
    )


def _make_deg_kernel(n_pad, r_t, g_chunk):
    nslice = n_pad // NS

    @functools.partial(
        pl.kernel,
        out_type=jax.ShapeDtypeStruct((NC, n_pad), jnp.float32),
        mesh=_sc_mesh(),
        compiler_params=pltpu.CompilerParams(use_tc_tiling_on_sc=False),
        scratch_types=[
            pltpu.VMEM((g_chunk, LANE), jnp.int32),
            pltpu.VMEM((LANE,), jnp.float32),
            pltpu.VMEM((nslice,), jnp.float32),
            pltpu.SemaphoreType.DMA,
            pltpu.VMEM_SHARED((n_pad,), jnp.float32),
        ],
    )
    def deg_kernel(dstm, zeros1, degp, dstb, onesb, bounce, ssem, shacc):
        c = lax.axis_index("c")
        s = lax.axis_index("s")
        t = c * NS + s
        sl = pl.ds(s * nslice, nslice)
        pltpu.sync_copy(zeros1.at[sl], bounce)
        pltpu.sync_copy(bounce, shacc.at[sl])
        for i in range(LANE // 16):
            onesb[pl.ds(i * 16, 16)] = jnp.ones((16,), jnp.float32)
        plsc.subcore_barrier()
        row0 = t * r_t

        def chunk(i, carry):
            base = row0 + i * g_chunk
            pltpu.sync_copy(dstm.at[pl.ds(base, g_chunk)], dstb)
            sd = [
                pltpu.async_copy(onesb, shacc.at[dstb.at[g]], ssem, add=True)
                for g in range(g_chunk)
            ]
            for d in sd:
                d.wait()
            return carry

        lax.fori_loop(0, r_t // g_chunk, chunk, 0)
        plsc.subcore_barrier()
        pltpu.sync_copy(shacc.at[sl], bounce)
        pltpu.sync_copy(bounce, degp.at[c].at[sl])

    return deg_kernel


def _make_edge_kernel(n_pad, F, r_t, g_chunk):
    nslice = n_pad // NS

    @functools.partial(
        pl.kernel,
        out_type=jax.ShapeDtypeStruct((NC, n_pad, F), jnp.float32),
        mesh=_sc_mesh(),
        compiler_params=pltpu.CompilerParams(use_tc_tiling_on_sc=False),
        scratch_types=[
            pltpu.VMEM((g_chunk, LANE), jnp.int32),
            pltpu.VMEM((g_chunk, LANE), jnp.int32),
            pltpu.VMEM((g_chunk, LANE, F), jnp.float32),
            pltpu.VMEM((nslice // 4, F), jnp.float32),
            pltpu.SemaphoreType.DMA,
            pltpu.SemaphoreType.DMA,
            pltpu.VMEM_SHARED((n_pad, F), jnp.float32),
        ],
    )
    def edge_kernel(gtab, srcm, dstm, zeros, outp, srcb, dstb, rowb, bounce,
                    gsem, ssem, shacc):
        c = lax.axis_index("c")
        s = lax.axis_index("s")
        t = c * NS + s
        q = nslice // 4
        for j in range(4):
            qs = pl.ds(s * nslice + j * q, q)
            pltpu.sync_copy(zeros.at[qs], bounce)
            pltpu.sync_copy(bounce, shacc.at[qs])
        plsc.subcore_barrier()
        row0 = t * r_t

        def chunk(i, carry):
            base = row0 + i * g_chunk
            pltpu.sync_copy(srcm.at[pl.ds(base, g_chunk)], srcb)
            pltpu.sync_copy(dstm.at[pl.ds(base, g_chunk)], dstb)
            gd = [
                pltpu.async_copy(gtab.at[srcb.at[g]], rowb.at[g], gsem)
                for g in range(g_chunk)
            ]
            for d in gd:
                d.wait()
            sd = [
                pltpu.async_copy(rowb.at[g], shacc.at[dstb.at[g]], ssem, add=True)
                for g in range(g_chunk)
            ]
            for d in sd:
                d.wait()
            return carry

        lax.fori_loop(0, r_t // g_chunk, chunk, 0)
        plsc.subcore_barrier()
        for j in range(4):
            qs = pl.ds(s * nslice + j * q, q)
            pltpu.sync_copy(shacc.at[qs], bounce)
            pltpu.sync_copy(bounce, outp.at[c].at[qs])

    return edge_kernel


def _row_spec(R, F):
    return pl.BlockSpec((R, F), lambda i: (i, 0))


def _full_spec(shape):
    return pl.BlockSpec(shape, lambda i: tuple(0 for _ in shape))


def _tc1(degp0, degp1, x_pad, W1, n_pad, R):
    def body(p0, p1, xr, w1, dinv_o, y1_o):
        deg = p0[...] + p1[...] + 1.0
        dinv = lax.rsqrt(deg)
        dinv_o[...] = dinv
        y1_o[...] = dinv * jnp.dot(xr[...], w1[...])

    return pl.pallas_call(
        body,
        grid=(n_pad // R,),
        in_specs=[_row_spec(R, 1), _row_spec(R, 1), _row_spec(R, 2),
                  _full_spec((2, 16))],
        out_specs=[_row_spec(R, 1), _row_spec(R, 16)],
        out_shape=[
            jax.ShapeDtypeStruct((n_pad, 1), jnp.float32),
            jax.ShapeDtypeStruct((n_pad, 16), jnp.float32),
        ],
    )(degp0, degp1, x_pad, W1)


def _tc2(dinv, y1, a0, a1, W2, b1, n_pad, R):
    def body(dv, y1r, a0r, a1r, w2a, w2b, b1r, y2a_o, y2b_o):
        d = dv[...]
        h1 = jnp.maximum(d * (a0r[...] + a1r[...] + y1r[...]) + b1r[...], 0.0)
        y2a_o[...] = d * jnp.dot(h1, w2a[...])
        y2b_o[...] = d * jnp.dot(h1, w2b[...])

    return pl.pallas_call(
        body,
        grid=(n_pad // R,),
        in_specs=[
            _row_spec(R, 1),
            _row_spec(R, 16),
            _row_spec(R, 16),
            _row_spec(R, 16),
            _full_spec((16, 16)),
            _full_spec((16, 16)),
            _full_spec((1, 16)),
        ],
        out_specs=[_row_spec(R, 16), _row_spec(R, 16)],
        out_shape=[
            jax.ShapeDtypeStruct((n_pad, 16), jnp.float32),
            jax.ShapeDtypeStruct((n_pad, 16), jnp.float32),
        ],
    )(dinv, y1, a0, a1, W2[:, :16], W2[:, 16:], b1.reshape(1, 16))


def _tc3(dinv, y2a, y2b, aA0, aA1, aB0, aB1, b2, W3, b3, W4, b4, n_pad, R):
    def body(dv, y2ar, y2br, a0r, a1r, b0r, b1r_, b2r, w3, b3r, w4, b4r, out_o):
        d = dv[...]
        za = d * (a0r[...] + a1r[...] + y2ar[...])
        zb = d * (b0r[...] + b1r_[...] + y2br[...])
        h2 = jnp.maximum(jnp.concatenate([za, zb], axis=1) + b2r[...], 0.0)
        h3 = jnp.dot(h2, w3[...]) + b3r[...]
        out_o[...] = jnp.dot(h3, w4[...]) + b4r[...]

    return pl.pallas_call(
        body,
        grid=(n_pad // R,),
        in_specs=[
            _row_spec(R, 1),
            _row_spec(R, 16),
            _row_spec(R, 16),
            _row_spec(R, 16),
            _row_spec(R, 16),
            _row_spec(R, 16),
            _row_spec(R, 16),
            _full_spec((1, 32)),
            _full_spec((32, 16)),
            _full_spec((1, 16)),
            _full_spec((16, 1)),
            _full_spec((1, 1)),
        ],
        out_specs=_row_spec(R, 1),
        out_shape=jax.ShapeDtypeStruct((n_pad, 1), jnp.float32),
    )(dinv, y2a, y2b, aA0, aA1, aB0, aB1, b2.reshape(1, 32), W3,
      b3.reshape(1, 16), W4, b4.reshape(1, 1))


def kernel(x, edge_index, W1, b1, W2, b2, W3, b3, W4, b4):
    N = x.shape[0]
    E = edge_index.shape[1]
    n_pad = -(-N // (NS * LANE)) * (NS * LANE)
    if n_pad == N:
        n_pad += NS * LANE
    r_t, g_chunk = _edge_geometry(E)
    e_pad = r_t * LANE * NC * NS

    src = edge_index[0]
    dst = edge_index[1]
    pad = e_pad - E
    if pad:
        ar = jnp.arange(pad, dtype=edge_index.dtype)
        src = jnp.concatenate([src, ar % N])
        dst = jnp.concatenate([dst, N + ar % (n_pad - N)])
    srcm = src.reshape(-1, LANE)
    dstm = dst.reshape(-1, LANE)

    x_pad = jnp.zeros((n_pad, 2), jnp.float32).at[:N].set(x)
    zeros1 = jnp.zeros((n_pad,), jnp.float32)
    zeros16 = jnp.zeros((n_pad, 16), jnp.float32)

    degp = _make_deg_kernel(n_pad, r_t, g_chunk)(dstm, zeros1)
    R = n_pad // 16
    dinv, y1 = _tc1(
        degp[0].reshape(n_pad, 1), degp[1].reshape(n_pad, 1), x_pad, W1, n_pad, R
    )

    edge16 = _make_edge_kernel(n_pad, 16, r_t, g_chunk)
    a1p = edge16(y1, srcm, dstm, zeros16)
    y2a, y2b = _tc2(dinv, y1, a1p[0], a1p[1], W2, b1, n_pad, R)

    a2A = edge16(y2a, srcm, dstm, zeros16)
    a2B = edge16(y2b, srcm, dstm, zeros16)
    out = _tc3(dinv, y2a, y2b, a2A[0], a2A[1], a2B[0], a2B[1], b2, W3, b3,
               W4, b4, n_pad, R)

    return out[:N]
